# Initial kernel scaffold; baseline (speedup 1.0000x reference)
import jax
import jax.numpy as jnp
from jax.experimental import pallas as pl


def kernel(z, W_shared_text, W_shared_graph, W_text, W_graph):
    raise NotImplementedError("write your pallas kernel here")


# speedup vs baseline: 12.4984x; 12.4984x over previous
import functools

import jax
import jax.numpy as jnp
from jax.experimental import pallas as pl

TOPK = 10
BETA = 0.25


def _l2n(x):
    n = jnp.sqrt(jnp.sum(x * x, axis=1, keepdims=True))
    return x / jnp.clip(n, 1e-12)


def _norm_books_kernel(a_ref, b_ref, c_ref, d_ref, an_ref, bn_ref, cn_ref, dn_ref):
    an_ref[...] = _l2n(a_ref[...])
    bn_ref[...] = _l2n(b_ref[...])
    cn_ref[...] = _l2n(c_ref[...])
    dn_ref[...] = _l2n(d_ref[...])


def _topk_weights(s):
    sm = s
    m0 = None
    t = None
    for k in range(TOPK):
        m = jnp.max(sm, axis=1, keepdims=True)
        if k == 0:
            m0 = m
        t = m
        if k < TOPK - 1:
            sm = jnp.where(sm == m, -jnp.inf, sm)
    p = jnp.where(s >= t, jnp.exp(s - m0), 0.0)
    z = jnp.sum(p, axis=1, keepdims=True)
    pres = (jnp.max(p, axis=0, keepdims=True) > 0.0).astype(jnp.float32)
    return p, z, pres


def _mmt(a, b_t):
    return jax.lax.dot_general(a, b_t, (((1,), (1,)), ((), ())),
                               preferred_element_type=jnp.float32)


def _shared_kernel(z_ref, est_ref, esg_ref,
                   zq_ref, ss_ref, pres_ref, *, d_half):
    i = pl.program_id(0)

    @pl.when(i == 0)
    def _init():
        ss_ref[...] = jnp.zeros_like(ss_ref)
        pres_ref[...] = jnp.zeros_like(pres_ref)

    zb = z_ref[...]
    zt = zb[:, :d_half]
    zg = zb[:, d_half:]
    est = est_ref[...]
    esg = esg_ref[...]
    s = 2.0 * (_mmt(_l2n(zt), est) + _mmt(_l2n(zg), esg))
    p, zden, pres = _topk_weights(s)
    zq_l = jnp.dot(p, est, preferred_element_type=jnp.float32)
    zq_r = jnp.dot(p, esg, preferred_element_type=jnp.float32)
    zq = jnp.concatenate([zq_l, zq_r], axis=1) / zden
    zq_ref[...] = zb + (zq - zb)
    ss_ref[...] += jnp.sum((zq - zb) ** 2).reshape(1, 1)
    pres_ref[...] = jnp.maximum(pres_ref[...], pres)


def _specific_kernel(z_ref, eb_ref, zq_ref, ss_ref, pres_ref,
                     *, d_half, half):
    i = pl.program_id(0)

    @pl.when(i == 0)
    def _init():
        ss_ref[...] = jnp.zeros_like(ss_ref)
        pres_ref[...] = jnp.zeros_like(pres_ref)

    zb = z_ref[...]
    zh = zb[:, :d_half] if half == 0 else zb[:, d_half:]
    eb = eb_ref[...]
    s = 2.0 * _mmt(_l2n(zh), eb)
    p, zden, pres = _topk_weights(s)
    zq = jnp.dot(p, eb, preferred_element_type=jnp.float32) / zden
    zq_ref[...] = zh + (zq - zh)
    ss_ref[...] += jnp.sum((zq - zh) ** 2).reshape(1, 1)
    pres_ref[...] = jnp.maximum(pres_ref[...], pres)


def kernel(z, W_shared_text, W_shared_graph, W_text, W_graph):
    n, d = z.shape
    n_e, d_half = W_text.shape
    f32 = jnp.float32

    nb_blk = min(1024, n_e)
    books = pl.pallas_call(
        _norm_books_kernel,
        grid=(n_e // nb_blk,),
        in_specs=[pl.BlockSpec((nb_blk, d_half), lambda i: (i, 0))] * 4,
        out_specs=[pl.BlockSpec((nb_blk, d_half), lambda i: (i, 0))] * 4,
        out_shape=[jax.ShapeDtypeStruct((n_e, d_half), f32)] * 4,
    )(W_shared_text, W_shared_graph, W_text, W_graph)
    estn, esgn, etn, egn = books

    tb = min(256, n)
    grid = (n // tb,)
    z_spec = pl.BlockSpec((tb, d), lambda i: (i, 0))
    book_spec = pl.BlockSpec((n_e, d_half), lambda i: (0, 0))
    acc_spec = pl.BlockSpec((1, 1), lambda i: (0, 0))
    pres_spec = pl.BlockSpec((1, n_e), lambda i: (0, 0))

    zq_sh, ss_sh, pres_sh = pl.pallas_call(
        functools.partial(_shared_kernel, d_half=d_half),
        grid=grid,
        in_specs=[z_spec, book_spec, book_spec],
        out_specs=[pl.BlockSpec((tb, d), lambda i: (i, 0)), acc_spec, pres_spec],
        out_shape=[jax.ShapeDtypeStruct((n, d), f32),
                   jax.ShapeDtypeStruct((1, 1), f32),
                   jax.ShapeDtypeStruct((1, n_e), f32)],
    )(z, estn, esgn)

    def specific(book, half):
        return pl.pallas_call(
            functools.partial(_specific_kernel, d_half=d_half, half=half),
            grid=grid,
            in_specs=[z_spec, book_spec],
            out_specs=[pl.BlockSpec((tb, d_half), lambda i: (i, 0)),
                       acc_spec, pres_spec],
            out_shape=[jax.ShapeDtypeStruct((n, d_half), f32),
                       jax.ShapeDtypeStruct((1, 1), f32),
                       jax.ShapeDtypeStruct((1, n_e), f32)],
        )(z, book)

    zq_t, ss_t, pres_t = specific(etn, 0)
    zq_g, ss_g, pres_g = specific(egn, 1)

    zt = z[:, :d_half]
    zg = z[:, d_half:]
    vq_sh = ss_sh[0, 0] / (n * d)
    vq_t = ss_t[0, 0] / (n * d_half)
    vq_g = ss_g[0, 0] / (n * d_half)

    def usage(pres):
        return (jnp.sum(pres) + (1.0 - pres[0, 0])) / n_e

    return (zq_sh, zq_t, zq_g, zt, zg,
            vq_sh, BETA * vq_sh, vq_t, BETA * vq_t, vq_g, BETA * vq_g,
            usage(pres_sh), usage(pres_t), usage(pres_g))
